# baseline (speedup 1.0000x reference)
import functools

import jax
import jax.numpy as jnp
from jax import lax
from jax.experimental import pallas as pl
from jax.experimental.pallas import tpu as pltpu
from jax.experimental.pallas import tpu_sc as plsc

WEIGHT = 5.0


def _sc_gather(table, idx):
    B = idx.shape[0]
    E = table.shape[1]
    info = plsc.get_sparse_core_info()
    nc, ns = info.num_cores, info.num_subcores
    nw = nc * ns
    b_per_w = B // nw
    mesh = plsc.VectorSubcoreMesh(core_axis_name="c", subcore_axis_name="s")

    @functools.partial(
        pl.kernel,
        mesh=mesh,
        out_type=jax.ShapeDtypeStruct((B, E), jnp.float32),
        scratch_types=[
            pltpu.VMEM((b_per_w,), jnp.int32),
            pltpu.VMEM((b_per_w, E), jnp.float32),
            pltpu.SemaphoreType.DMA,
        ],
    )
    def gather_kernel(table_hbm, idx_hbm, out_hbm, idx_v, rows_v, sem):
        wid = lax.axis_index("s") * nc + lax.axis_index("c")
        base = wid * b_per_w
        pltpu.sync_copy(idx_hbm.at[pl.ds(base, b_per_w)], idx_v)
        pltpu.async_copy(table_hbm.at[idx_v], rows_v, sem).wait()
        pltpu.sync_copy(rows_v, out_hbm.at[pl.ds(base, b_per_w)])

    return gather_kernel(table, idx)


def _bias_body(x_ref, w_ref, o_ref):
    x = x_ref[...]
    w = w_ref[...]
    xs = ((2.0 * WEIGHT) * x).astype(jnp.bfloat16)
    wb = w.astype(jnp.bfloat16)
    t2 = lax.dot_general(xs, wb, (((1,), (1,)), ((), ())),
                         preferred_element_type=jnp.float32)
    negw = jnp.full((1, w.shape[1]), -WEIGHT, jnp.bfloat16)
    t1row = lax.dot_general(negw, wb * wb, (((1,), (1,)), ((), ())),
                            preferred_element_type=jnp.float32)
    t3 = jnp.sum(x * x, axis=1)
    o_ref[...] = t2 + t1row + (-WEIGHT) * t3[:, None]


def kernel(sampled_ids, embed_weight):
    Bt, S = sampled_ids.shape
    V, E = embed_weight.shape
    B = Bt * S
    idx = sampled_ids.reshape(B).astype(jnp.int32)

    cur = _sc_gather(embed_weight, idx)

    VT = 4352
    out = pl.pallas_call(
        _bias_body,
        grid=(pl.cdiv(V, VT),),
        compiler_params=pltpu.CompilerParams(
            vmem_limit_bytes=64 * 1024 * 1024,
            dimension_semantics=("parallel",)),
        in_specs=[
            pl.BlockSpec((B, E), lambda i: (0, 0)),
            pl.BlockSpec((VT, E), lambda i: (i, 0)),
        ],
        out_specs=pl.BlockSpec((B, VT), lambda i: (0, i)),
        out_shape=jax.ShapeDtypeStruct((B, V), jnp.float32),
    )(cur, embed_weight)

    return out.reshape(Bt, S, V)


# speedup vs baseline: 1.0445x; 1.0445x over previous
import functools

import jax
import jax.numpy as jnp
from jax import lax
from jax.experimental import pallas as pl
from jax.experimental.pallas import tpu as pltpu
from jax.experimental.pallas import tpu_sc as plsc

WEIGHT = 5.0


def _sc_gather(table, idx):
    B = idx.shape[0]
    E = table.shape[1]
    info = plsc.get_sparse_core_info()
    nc, ns = info.num_cores, info.num_subcores
    nw = nc * ns
    b_per_w = B // nw
    mesh = plsc.VectorSubcoreMesh(core_axis_name="c", subcore_axis_name="s")

    @functools.partial(
        pl.kernel,
        mesh=mesh,
        out_type=jax.ShapeDtypeStruct((B, E), jnp.float32),
        scratch_types=[
            pltpu.VMEM((b_per_w,), jnp.int32),
            pltpu.VMEM((b_per_w, E), jnp.float32),
            pltpu.SemaphoreType.DMA,
        ],
    )
    def gather_kernel(table_hbm, idx_hbm, out_hbm, idx_v, rows_v, sem):
        wid = lax.axis_index("s") * nc + lax.axis_index("c")
        base = wid * b_per_w
        pltpu.sync_copy(idx_hbm.at[pl.ds(base, b_per_w)], idx_v)
        pltpu.async_copy(table_hbm.at[idx_v], rows_v, sem).wait()
        pltpu.sync_copy(rows_v, out_hbm.at[pl.ds(base, b_per_w)])

    return gather_kernel(table, idx)


def _bias_body(x_ref, w_ref, o_ref):
    x = x_ref[...]
    w = w_ref[...]
    xs = ((2.0 * WEIGHT) * x).astype(jnp.bfloat16)
    wb = w.astype(jnp.bfloat16)
    t2 = lax.dot_general(xs, wb, (((1,), (1,)), ((), ())),
                         preferred_element_type=jnp.float32)
    acc = None
    for c in range(0, w.shape[1], 128):
        ws = w[:, c:c + 128]
        sq = ws * ws
        acc = sq if acc is None else acc + sq
    negw = jnp.full((1, 128), -WEIGHT, jnp.bfloat16)
    t1row = lax.dot_general(negw, acc.astype(jnp.bfloat16),
                            (((1,), (1,)), ((), ())),
                            preferred_element_type=jnp.float32)
    t3 = jnp.sum(x * x, axis=1)
    o_ref[...] = t2 + t1row + (-WEIGHT) * t3[:, None]


def kernel(sampled_ids, embed_weight):
    Bt, S = sampled_ids.shape
    V, E = embed_weight.shape
    B = Bt * S
    idx = sampled_ids.reshape(B).astype(jnp.int32)

    cur = _sc_gather(embed_weight, idx)

    VT = 4736
    out = pl.pallas_call(
        _bias_body,
        grid=(pl.cdiv(V, VT),),
        compiler_params=pltpu.CompilerParams(
            vmem_limit_bytes=64 * 1024 * 1024,
            dimension_semantics=("parallel",)),
        in_specs=[
            pl.BlockSpec((B, E), lambda i: (0, 0)),
            pl.BlockSpec((VT, E), lambda i: (i, 0)),
        ],
        out_specs=pl.BlockSpec((B, VT), lambda i: (0, i)),
        out_shape=jax.ShapeDtypeStruct((B, V), jnp.float32),
    )(cur, embed_weight)

    return out.reshape(Bt, S, V)


# speedup vs baseline: 1.0481x; 1.0034x over previous
import functools

import jax
import jax.numpy as jnp
from jax import lax
from jax.experimental import pallas as pl
from jax.experimental.pallas import tpu as pltpu
from jax.experimental.pallas import tpu_sc as plsc

WEIGHT = 5.0


def _sc_gather(table, idx):
    B = idx.shape[0]
    E = table.shape[1]
    info = plsc.get_sparse_core_info()
    nc, ns = info.num_cores, info.num_subcores
    nw = nc * ns
    b_per_w = B // nw
    mesh = plsc.VectorSubcoreMesh(core_axis_name="c", subcore_axis_name="s")

    @functools.partial(
        pl.kernel,
        mesh=mesh,
        out_type=jax.ShapeDtypeStruct((B, E), jnp.float32),
        scratch_types=[
            pltpu.VMEM((b_per_w,), jnp.int32),
            pltpu.VMEM((b_per_w, E), jnp.float32),
            pltpu.SemaphoreType.DMA,
        ],
    )
    def gather_kernel(table_hbm, idx_hbm, out_hbm, idx_v, rows_v, sem):
        wid = lax.axis_index("s") * nc + lax.axis_index("c")
        base = wid * b_per_w
        pltpu.sync_copy(idx_hbm.at[pl.ds(base, b_per_w)], idx_v)
        pltpu.async_copy(table_hbm.at[idx_v], rows_v, sem).wait()
        pltpu.sync_copy(rows_v, out_hbm.at[pl.ds(base, b_per_w)])

    return gather_kernel(table, idx)


def _bias_body(x_ref, w_ref, o_ref):
    x = x_ref[...]
    w = w_ref[...]
    xs = ((2.0 * WEIGHT) * x).astype(jnp.bfloat16)
    wb = w.astype(jnp.bfloat16)
    t2 = lax.dot_general(xs, wb, (((1,), (1,)), ((), ())),
                         preferred_element_type=jnp.float32)
    acc = None
    for c in range(0, w.shape[1], 128):
        ws = w[:, c:c + 128]
        sq = ws * ws
        acc = sq if acc is None else acc + sq
    negw = jnp.full((1, 128), -WEIGHT, jnp.bfloat16)
    t1row = lax.dot_general(negw, acc.astype(jnp.bfloat16),
                            (((1,), (1,)), ((), ())),
                            preferred_element_type=jnp.float32)
    t3 = jnp.sum(x * x, axis=1)
    o_ref[...] = t2 + t1row + (-WEIGHT) * t3[:, None]


def kernel(sampled_ids, embed_weight):
    Bt, S = sampled_ids.shape
    V, E = embed_weight.shape
    B = Bt * S
    idx = sampled_ids.reshape(B).astype(jnp.int32)

    cur = _sc_gather(embed_weight, idx)

    VT = 4864
    out = pl.pallas_call(
        _bias_body,
        grid=(pl.cdiv(V, VT),),
        compiler_params=pltpu.CompilerParams(
            vmem_limit_bytes=64 * 1024 * 1024,
            dimension_semantics=("parallel",)),
        in_specs=[
            pl.BlockSpec((B, E), lambda i: (0, 0)),
            pl.BlockSpec((VT, E), lambda i: (i, 0)),
        ],
        out_specs=pl.BlockSpec((B, VT), lambda i: (0, i)),
        out_shape=jax.ShapeDtypeStruct((B, V), jnp.float32),
    )(cur, embed_weight)

    return out.reshape(Bt, S, V)
